# Initial kernel scaffold; baseline (speedup 1.0000x reference)
import jax
import jax.numpy as jnp
from jax.experimental import pallas as pl


def kernel(x, edge_index, batch, num_graphs, token_x, W1, b1, W2, b2, Wp, bp):
    raise NotImplementedError("write your pallas kernel here")


# speedup vs baseline: 3.9894x; 3.9894x over previous
import functools
import numpy as np
import jax
import jax.numpy as jnp
from jax import lax
from jax.experimental import pallas as pl
from jax.experimental.pallas import tpu as pltpu

NN = 10000
T = 5
G = 64
NREAL = NN + T * G
NPAD = 10368
D = 128
BM = 384
NB = NPAD // BM
NNP = 10240


def _sim_body(tok_ref, x_ref, cross_ref, inner_ref):
    j = pl.program_id(0)
    tok = tok_ref[...]
    xb = x_ref[...]
    d = lax.dot_general(tok, xb, (((1,), (1,)), ((), ())),
                        preferred_element_type=jnp.float32)
    col = j * 1280 + lax.broadcasted_iota(jnp.int32, (8, 1280), 1)
    cross_ref[...] = jnp.where(col < NN, d, -1.0)

    @pl.when(j == 0)
    def _():
        i8 = lax.dot_general(tok, tok, (((1,), (1,)), ((), ())),
                             preferred_element_type=jnp.float32)
        inner_ref[...] = jnp.concatenate(
            [i8, jnp.zeros((8, 120), jnp.float32)], axis=1)


def _sim(tok_pad, x_padn):
    return pl.pallas_call(
        _sim_body,
        grid=(NNP // 1280,),
        in_specs=[
            pl.BlockSpec((8, 128), lambda j: (0, 0)),
            pl.BlockSpec((1280, 128), lambda j: (j, 0)),
        ],
        out_specs=[
            pl.BlockSpec((8, 1280), lambda j: (0, j)),
            pl.BlockSpec((8, 128), lambda j: (0, 0)),
        ],
        out_shape=[
            jax.ShapeDtypeStruct((8, NNP), jnp.float32),
            jax.ShapeDtypeStruct((8, 128), jnp.float32),
        ],
    )(tok_pad, x_padn)


def _deg_body(a_ref, dinv_ref, acc_ref):
    i = pl.program_id(0)
    j = pl.program_id(1)
    a = a_ref[...]
    rows = i * BM + lax.broadcasted_iota(jnp.int32, (BM, BM), 0)
    cols = j * BM + lax.broadcasted_iota(jnp.int32, (BM, BM), 1)
    a = jnp.where(cols >= NREAL, 0.0, a)
    a = jnp.where(rows == cols, 2.0, a)

    @pl.when(j == 0)
    def _():
        acc_ref[...] = jnp.zeros_like(acc_ref)

    acc_ref[...] += jnp.sum(a, axis=1, keepdims=True)

    @pl.when(j == NB - 1)
    def _():
        deg = acc_ref[...]
        dinv = jax.lax.rsqrt(jnp.maximum(deg, 1e-12))
        r = i * BM + lax.broadcasted_iota(jnp.int32, (BM, 1), 0)
        dinv_ref[...] = jnp.where(r < NREAL, dinv, 0.0)


def _deg(a_mat):
    return pl.pallas_call(
        _deg_body,
        grid=(NB, NB),
        in_specs=[pl.BlockSpec((BM, BM), lambda i, j: (i, j))],
        out_specs=pl.BlockSpec((BM, 1), lambda i, j: (i, 0)),
        out_shape=jax.ShapeDtypeStruct((NPAD, 1), jnp.float32),
        scratch_shapes=[pltpu.VMEM((BM, 1), jnp.float32)],
    )(a_mat)


def _y_body(h_ref, w_ref, dinv_ref, y_ref):
    y_ref[...] = lax.dot_general(
        h_ref[...], w_ref[...], (((1,), (0,)), ((), ())),
        preferred_element_type=jnp.float32) * dinv_ref[...]


def _y(h, w, dinv):
    return pl.pallas_call(
        _y_body,
        grid=(NB,),
        in_specs=[
            pl.BlockSpec((BM, D), lambda i: (i, 0)),
            pl.BlockSpec((D, D), lambda i: (0, 0)),
            pl.BlockSpec((BM, 1), lambda i: (i, 0)),
        ],
        out_specs=pl.BlockSpec((BM, D), lambda i: (i, 0)),
        out_shape=jax.ShapeDtypeStruct((NPAD, D), jnp.float32),
    )(h, w, dinv)


def _agg_body(a_ref, y_ref, dinv_ref, b_ref, out_ref, acc_ref, *, leaky):
    i = pl.program_id(0)
    j = pl.program_id(1)
    a = a_ref[...]
    rows = i * BM + lax.broadcasted_iota(jnp.int32, (BM, BM), 0)
    cols = j * BM + lax.broadcasted_iota(jnp.int32, (BM, BM), 1)
    a = jnp.where(rows == cols, 2.0, a)

    yb = y_ref[pl.ds(j * BM, BM), :]

    @pl.when(j == 0)
    def _():
        acc_ref[...] = jnp.zeros_like(acc_ref)

    acc_ref[...] += lax.dot_general(a, yb, (((1,), (0,)), ((), ())),
                                    preferred_element_type=jnp.float32)

    @pl.when(j == NB - 1)
    def _():
        o = acc_ref[...] * dinv_ref[...] + b_ref[...]
        if leaky:
            o = jnp.where(o >= 0, o, 0.01 * o)
        out_ref[...] = o


def _agg(a_mat, y, dinv, b2d, leaky):
    return pl.pallas_call(
        functools.partial(_agg_body, leaky=leaky),
        grid=(NB, NB),
        in_specs=[
            pl.BlockSpec((BM, BM), lambda i, j: (i, j)),
            pl.BlockSpec((NPAD, D), lambda i, j: (0, 0)),
            pl.BlockSpec((BM, 1), lambda i, j: (i, 0)),
            pl.BlockSpec((1, D), lambda i, j: (0, 0)),
        ],
        out_specs=pl.BlockSpec((BM, D), lambda i, j: (i, 0)),
        out_shape=jax.ShapeDtypeStruct((NPAD, D), jnp.float32),
        scratch_shapes=[pltpu.VMEM((BM, D), jnp.float32)],
    )(a_mat, y, dinv, b2d)


def _pool_body(emb_ref, bat_ref, wp_ref, bp_ref, out_ref, sum_ref, cnt_ref):
    i = pl.program_id(0)
    b = bat_ref[...]
    gids = lax.broadcasted_iota(jnp.int32, (64, BM), 0)
    p = jnp.where(gids == b, 1.0, 0.0)

    @pl.when(i == 0)
    def _():
        sum_ref[...] = jnp.zeros_like(sum_ref)
        cnt_ref[...] = jnp.zeros_like(cnt_ref)

    sum_ref[...] += lax.dot_general(p, emb_ref[...], (((1,), (0,)), ((), ())),
                                    preferred_element_type=jnp.float32)
    cnt_ref[...] += jnp.sum(p, axis=1, keepdims=True)

    @pl.when(i == NB - 1)
    def _():
        graph = sum_ref[...] / jnp.maximum(cnt_ref[...], 1.0)
        logits = lax.dot_general(graph, wp_ref[...], (((1,), (0,)), ((), ())),
                                 preferred_element_type=jnp.float32) + bp_ref[...]
        col = lax.broadcasted_iota(jnp.int32, (64, 128), 1)
        z = jnp.where(col < 2, logits, -1e30)
        m = jnp.max(z, axis=1, keepdims=True)
        e = jnp.where(col < 2, jnp.exp(z - m), 0.0)
        out_ref[...] = e / jnp.sum(e, axis=1, keepdims=True)


def _pool(emb, bat2d, wp_pad, bp_pad):
    return pl.pallas_call(
        _pool_body,
        grid=(NB,),
        in_specs=[
            pl.BlockSpec((BM, D), lambda i: (i, 0)),
            pl.BlockSpec((1, BM), lambda i: (0, i)),
            pl.BlockSpec((D, D), lambda i: (0, 0)),
            pl.BlockSpec((1, D), lambda i: (0, 0)),
        ],
        out_specs=pl.BlockSpec((64, 128), lambda i: (0, 0)),
        out_shape=jax.ShapeDtypeStruct((64, 128), jnp.float32),
        scratch_shapes=[pltpu.VMEM((64, D), jnp.float32),
                        pltpu.VMEM((64, 1), jnp.float32)],
    )(emb, bat2d, wp_pad, bp_pad)


def _build_a(edge_index, batch, cross_dot, inner_dot):
    src = edge_index[0]
    dst = edge_index[1]
    rows = [dst, src]
    cols = [src, dst]
    cmask = cross_dot[:T, :NN] >= 0.0
    gtok = (NN + T * batch[None, :]
            + jnp.arange(T, dtype=jnp.int32)[:, None])
    node = jnp.broadcast_to(jnp.arange(NN, dtype=jnp.int32)[None, :], (T, NN))
    gtok_m = jnp.where(cmask, gtok, NPAD).reshape(-1)
    node_m = jnp.where(cmask, node, NPAD).reshape(-1)
    rows += [gtok_m, node_m]
    cols += [node_m, gtok_m]
    imask = inner_dot[:T, :T] >= 0.0
    offs = NN + T * jnp.arange(G, dtype=jnp.int32)
    r = jnp.arange(T, dtype=jnp.int32)[:, None]
    c = jnp.arange(T, dtype=jnp.int32)[None, :]
    ir = jnp.where(imask, offs[:, None, None] + r[None], NPAD).reshape(-1)
    ic = jnp.where(imask, offs[:, None, None] + c[None], NPAD).reshape(-1)
    rows.append(ir)
    cols.append(ic)
    rows = jnp.concatenate(rows)
    cols = jnp.concatenate(cols)
    a = jnp.zeros((NPAD, NPAD), jnp.float32)
    a = a.at[rows, cols].set(1.0, mode="drop")
    return a


def kernel(x, edge_index, batch, num_graphs, token_x, W1, b1, W2, b2, Wp, bp):
    del num_graphs
    f32 = jnp.float32
    tok_pad = jnp.concatenate([token_x, jnp.zeros((3, D), f32)], axis=0)
    x_padn = jnp.concatenate([x, jnp.zeros((NNP - NN, D), f32)], axis=0)
    cross_dot, inner_dot = _sim(tok_pad, x_padn)

    a_mat = _build_a(edge_index, batch, cross_dot, inner_dot)

    dinv = _deg(a_mat)

    x_aug = jnp.concatenate(
        [x, jnp.tile(token_x, (G, 1)), jnp.zeros((NPAD - NREAL, D), f32)],
        axis=0)
    b1_2d = b1.reshape(1, D)
    b2_2d = b2.reshape(1, D)

    y1 = _y(x_aug, W1, dinv)
    h1 = _agg(a_mat, y1, dinv, b1_2d, leaky=True)
    y2 = _y(h1, W2, dinv)
    emb = _agg(a_mat, y2, dinv, b2_2d, leaky=False)

    token_batch = np.repeat(np.arange(G, dtype=np.int32), T)
    pad_batch = np.full((NPAD - NREAL,), -1, np.int32)
    bat2d = jnp.concatenate(
        [batch, jnp.asarray(token_batch), jnp.asarray(pad_batch)]
    ).reshape(1, NPAD)
    wp_pad = jnp.concatenate([Wp, jnp.zeros((D, D - 2), f32)], axis=1)
    bp_pad = jnp.concatenate([bp, jnp.zeros((D - 2,), f32)]).reshape(1, D)

    out = _pool(emb, bat2d, wp_pad, bp_pad)
    return out[:, :2]
